# baseline (speedup 1.0000x reference)
import functools

import jax
import jax.numpy as jnp
from jax import lax
from jax.experimental import pallas as pl
from jax.experimental.pallas import tpu as pltpu
from jax.experimental.pallas import tpu_sc as plsc

_E = 16
_TOPK = 4
_D = 512
_H = 1365
_T = 8192
_BM = 256
_HEPAD = 20480
_HTILES = _HEPAD // _BM
_HLAST = _HEPAD - 1

_NW = 32


def _router_body(x_ref, gw_ref, b_ref, tri_ref, probs_ref, pv_ref):
    x = x_ref[...]
    logits = jnp.dot(x, gw_ref[...], preferred_element_type=jnp.float32)
    logits = logits + b_ref[...]
    lane = lax.broadcasted_iota(jnp.int32, logits.shape, 1)
    work = logits
    chosen = jnp.zeros(logits.shape, dtype=jnp.bool_)
    for _ in range(_TOPK):
        m = jnp.max(work, axis=-1, keepdims=True)
        is_m = work == m
        first = jnp.min(jnp.where(is_m, lane, _E), axis=-1, keepdims=True)
        sel = lane == first
        chosen = jnp.logical_or(chosen, sel)
        work = jnp.where(sel, -jnp.inf, work)
    mx = jnp.max(logits, axis=-1, keepdims=True)
    ex = jnp.where(chosen, jnp.exp(logits - mx), 0.0)
    denom = jnp.sum(ex, axis=-1, keepdims=True)
    probs_ref[...] = ex / denom
    rank = jnp.dot(chosen.astype(jnp.float32), tri_ref[...],
                   preferred_element_type=jnp.float32)
    cols = [jnp.sum(jnp.where(chosen & (rank == k + 1), ex, 0.0),
                    axis=-1, keepdims=True) / denom for k in range(_TOPK)]
    zero = jnp.zeros_like(cols[0])
    pv_ref[...] = jnp.concatenate(cols + [zero] * (8 - _TOPK), axis=-1)


def _router(x2d, gate_w, bias, tri):
    bt = 512
    return pl.pallas_call(
        _router_body,
        grid=(_T // bt,),
        in_specs=[
            pl.BlockSpec((bt, _D), lambda i: (i, 0)),
            pl.BlockSpec((_D, _E), lambda i: (0, 0)),
            pl.BlockSpec((1, _E), lambda i: (0, 0)),
            pl.BlockSpec((_E, _E), lambda i: (0, 0)),
        ],
        out_specs=[
            pl.BlockSpec((bt, _E), lambda i: (i, 0)),
            pl.BlockSpec((bt, 8), lambda i: (i, 0)),
        ],
        out_shape=[
            jax.ShapeDtypeStruct((_T, _E), jnp.float32),
            jax.ShapeDtypeStruct((_T, 8), jnp.float32),
        ],
    )(x2d, gate_w, bias.reshape(1, _E), tri)


def _sc_scatter(src, didx4, epad):
    t, d = src.shape
    nk = didx4.shape[0]
    per_w = t // _NW
    chunk = 64
    n_chunks = per_w // chunk
    mesh = plsc.VectorSubcoreMesh(core_axis_name="c", subcore_axis_name="s")

    @functools.partial(
        pl.kernel,
        mesh=mesh,
        out_type=jax.ShapeDtypeStruct((epad, d), jnp.float32),
        scratch_types=[
            pltpu.VMEM((chunk, d), jnp.float32),
            pltpu.VMEM((chunk, d), jnp.float32),
            pltpu.VMEM((nk, n_chunks, chunk), jnp.int32),
            pltpu.SemaphoreType.DMA,
            pltpu.SemaphoreType.DMA,
            pltpu.SemaphoreType.DMA,
            pltpu.SemaphoreType.DMA,
        ],
    )
    def sk(src_hbm, didx_hbm, out_hbm, buf0, buf1, idx3, rs0, rs1, ws0, ws1):
        wid = lax.axis_index("s") * 2 + lax.axis_index("c")
        base = pl.multiple_of(wid * per_w, 8)
        bufs = (buf0, buf1)
        rsems = (rs0, rs1)
        wsems = (ws0, ws1)

        def read(c):
            off = pl.multiple_of(base + c * chunk, 8)
            return pltpu.async_copy(
                src_hbm.at[pl.ds(off, chunk)], bufs[c % 2], rsems[c % 2])

        for k in range(nk):
            pltpu.sync_copy(didx_hbm.at[k, wid], idx3.at[k])
        read(0)
        read(1)
        for c in range(n_chunks):
            off = pl.multiple_of(base + c * chunk, 8)
            pltpu.make_async_copy(
                src_hbm.at[pl.ds(off, chunk)], bufs[c % 2],
                rsems[c % 2]).wait()
            handles = [
                pltpu.async_copy(bufs[c % 2], out_hbm.at[idx3.at[k, c]],
                                 wsems[c % 2])
                for k in range(nk)
            ]
            for h in handles:
                h.wait()
            if c + 2 < n_chunks:
                read(c + 2)

    return sk(src, didx4)


def _sc_gather(src, idx, chunk):
    m, d = idx.shape[0], src.shape[1]
    per_w = m // _NW
    n_chunks = per_w // chunk
    mesh = plsc.VectorSubcoreMesh(core_axis_name="c", subcore_axis_name="s")

    @functools.partial(
        pl.kernel,
        mesh=mesh,
        out_type=jax.ShapeDtypeStruct((m, d), jnp.float32),
        scratch_types=[
            pltpu.VMEM((per_w,), jnp.int32),
            pltpu.VMEM((chunk, d), jnp.float32),
            pltpu.VMEM((chunk, d), jnp.float32),
            pltpu.SemaphoreType.DMA,
            pltpu.SemaphoreType.DMA,
        ],
    )
    def gk(src_hbm, idx_hbm, out_hbm, idx_v, buf0, buf1, sem0, sem1):
        wid = lax.axis_index("s") * 2 + lax.axis_index("c")
        base = pl.multiple_of(wid * per_w, 8)
        pltpu.sync_copy(idx_hbm.at[pl.ds(base, per_w)], idx_v)

        def start(j, buf, sem):
            off = pl.multiple_of(j * chunk, 8)
            return pltpu.async_copy(
                src_hbm.at[idx_v.at[pl.ds(off, chunk)]], buf, sem)

        def finish(j, buf, sem):
            ioff = pl.multiple_of(j * chunk, 8)
            pltpu.make_async_copy(
                src_hbm.at[idx_v.at[pl.ds(ioff, chunk)]], buf, sem).wait()
            off = pl.multiple_of(base + j * chunk, 8)
            pltpu.sync_copy(buf, out_hbm.at[pl.ds(off, chunk)])

        start(0, buf0, sem0)

        def body(jj, carry):
            j0 = jj * 2

            @pl.when(j0 + 1 < n_chunks)
            def _():
                start(j0 + 1, buf1, sem1)

            finish(j0, buf0, sem0)

            @pl.when(j0 + 2 < n_chunks)
            def _():
                start(j0 + 2, buf0, sem0)

            @pl.when(j0 + 1 < n_chunks)
            def _():
                finish(j0 + 1, buf1, sem1)

            return carry

        lax.fori_loop(0, (n_chunks + 1) // 2, body, 0)

    return gk(src, idx)


def _expert_body(eot_ref, x_ref, w1_ref, w2_ref, w3_ref, y_ref):
    xb = x_ref[...].astype(jnp.bfloat16)
    h = jnp.dot(xb, w1_ref[0], preferred_element_type=jnp.float32)
    g = h * jax.nn.sigmoid(h)
    v = jnp.dot(xb, w2_ref[0], preferred_element_type=jnp.float32)
    gv = (g * v).astype(jnp.bfloat16)
    y_ref[...] = jnp.dot(gv, w3_ref[0], preferred_element_type=jnp.float32)


def _grouped_experts(exp_tile, xs, w1, w2, w3):
    def wmap(i, eot):
        return (eot[i], 0, 0)

    grid_spec = pltpu.PrefetchScalarGridSpec(
        num_scalar_prefetch=1,
        grid=(xs.shape[0] // _BM,),
        in_specs=[
            pl.BlockSpec((_BM, _D), lambda i, eot: (i, 0)),
            pl.BlockSpec((1, _D, _H), wmap),
            pl.BlockSpec((1, _D, _H), wmap),
            pl.BlockSpec((1, _H, _D), wmap),
        ],
        out_specs=pl.BlockSpec((_BM, _D), lambda i, eot: (i, 0)),
    )
    return pl.pallas_call(
        _expert_body,
        grid_spec=grid_spec,
        out_shape=jax.ShapeDtypeStruct((xs.shape[0], _D), jnp.float32),
    )(exp_tile, xs, w1, w2, w3)


def _shared_body(x_ref, w1_ref, w2_ref, w3_ref, y_ref):
    xb = x_ref[...].astype(jnp.bfloat16)
    h = jnp.dot(xb, w1_ref[...], preferred_element_type=jnp.float32)
    g = h * jax.nn.sigmoid(h)
    v = jnp.dot(xb, w2_ref[...], preferred_element_type=jnp.float32)
    gv = (g * v).astype(jnp.bfloat16)
    y_ref[...] = jnp.dot(gv, w3_ref[...], preferred_element_type=jnp.float32)


def _shared_expert(x2d, sw1, sw2, sw3):
    return pl.pallas_call(
        _shared_body,
        grid=(_T // _BM,),
        in_specs=[
            pl.BlockSpec((_BM, _D), lambda i: (i, 0)),
            pl.BlockSpec((_D, _H), lambda i: (0, 0)),
            pl.BlockSpec((_D, _H), lambda i: (0, 0)),
            pl.BlockSpec((_H, _D), lambda i: (0, 0)),
        ],
        out_specs=pl.BlockSpec((_BM, _D), lambda i: (i, 0)),
        out_shape=jax.ShapeDtypeStruct((_T, _D), jnp.float32),
    )(x2d, sw1, sw2, sw3)


def _combine_body(za_ref, zb_ref, ysh_ref, pv_ref, o_ref):
    za = za_ref[...]
    zb = zb_ref[...]
    pv = pv_ref[...]
    acc = ysh_ref[...]
    acc = acc + za[0] * pv[:, 0:1] + za[1] * pv[:, 1:2]
    acc = acc + zb[0] * pv[:, 2:3] + zb[1] * pv[:, 3:4]
    o_ref[...] = acc


def _combine(za, zb, y, pv):
    bc = 512
    return pl.pallas_call(
        _combine_body,
        grid=(_T // bc,),
        in_specs=[
            pl.BlockSpec((2, bc, _D), lambda i: (0, i, 0)),
            pl.BlockSpec((2, bc, _D), lambda i: (0, i, 0)),
            pl.BlockSpec((bc, _D), lambda i: (i, 0)),
            pl.BlockSpec((bc, 8), lambda i: (i, 0)),
        ],
        out_specs=pl.BlockSpec((bc, _D), lambda i: (i, 0)),
        out_shape=jax.ShapeDtypeStruct((_T, _D), jnp.float32),
    )(za, zb, y, pv)


def kernel(x, gate_w, w1, w2, w3, sw1, sw2, sw3, routing_bias):
    b, s, _ = x.shape
    x2d = x.reshape(_T, _D)

    tri = jnp.triu(jnp.ones((_E, _E), jnp.float32))
    probs, pv = _router(x2d, gate_w, routing_bias, tri)

    mask = probs > 0.0
    maski = mask.astype(jnp.int32)
    rank_in_row = jnp.cumsum(maski, axis=1) - 1
    halves = []
    for h in range(2):
        maskh = mask & (rank_in_row >= 2 * h) & (rank_in_row < 2 * h + 2)
        mih = maskh.astype(jnp.int32)
        counts = jnp.sum(mih, axis=0)
        padded = ((counts + _BM - 1) // _BM) * _BM
        ends = jnp.cumsum(padded)
        starts = ends - padded
        rank = jnp.cumsum(mih, axis=0) - 1
        destf = jnp.where(maskh, starts[None, :] + rank, 0)
        rirh = jnp.cumsum(mih, axis=1) - 1
        nrow = jnp.sum(mih, axis=1)
        dest2 = [jnp.where(
            nrow > k,
            jnp.sum(jnp.where(maskh & (rirh == k), destf, 0), axis=1),
            _HLAST) for k in range(2)]
        didx = jnp.stack(dest2)
        exp_tile = jnp.repeat(jnp.arange(_E, dtype=jnp.int32), padded // _BM,
                              total_repeat_length=_HTILES)
        halves.append((didx.reshape(2, _NW, -1, 64), didx.reshape(-1),
                       exp_tile))

    w1b = w1.astype(jnp.bfloat16)
    w2b = w2.astype(jnp.bfloat16)
    w3b = w3.astype(jnp.bfloat16)
    ysh = _shared_expert(x2d, sw1.astype(jnp.bfloat16),
                         sw2.astype(jnp.bfloat16), sw3.astype(jnp.bfloat16))
    xsa = _sc_scatter(x2d, halves[0][0], _HEPAD)
    ya = _grouped_experts(halves[0][2], xsa, w1b, w2b, w3b)
    xsb = _sc_scatter(x2d, halves[1][0], _HEPAD)
    yb = _grouped_experts(halves[1][2], xsb, w1b, w2b, w3b)
    za = _sc_gather(ya, halves[0][1], chunk=64)
    zb = _sc_gather(yb, halves[1][1], chunk=64)
    out2d = _combine(za.reshape(2, _T, _D), zb.reshape(2, _T, _D), ysh, pv)
    return out2d.reshape(b, s, _D)


# speedup vs baseline: 1.0865x; 1.0865x over previous
import functools

import jax
import jax.numpy as jnp
from jax import lax
from jax.experimental import pallas as pl
from jax.experimental.pallas import tpu as pltpu
from jax.experimental.pallas import tpu_sc as plsc

_E = 16
_TOPK = 4
_D = 512
_H = 1365
_T = 8192
_BM = 256
_EPAD = 36864
_NTILES = _EPAD // _BM
_LAST = _EPAD - 1

_NW = 32


def _router_body(x_ref, gw_ref, b_ref, tri_ref, probs_ref, pv_ref):
    x = x_ref[...]
    logits = jnp.dot(x, gw_ref[...], preferred_element_type=jnp.float32)
    logits = logits + b_ref[...]
    lane = lax.broadcasted_iota(jnp.int32, logits.shape, 1)
    work = logits
    chosen = jnp.zeros(logits.shape, dtype=jnp.bool_)
    for _ in range(_TOPK):
        m = jnp.max(work, axis=-1, keepdims=True)
        is_m = work == m
        first = jnp.min(jnp.where(is_m, lane, _E), axis=-1, keepdims=True)
        sel = lane == first
        chosen = jnp.logical_or(chosen, sel)
        work = jnp.where(sel, -jnp.inf, work)
    mx = jnp.max(logits, axis=-1, keepdims=True)
    ex = jnp.where(chosen, jnp.exp(logits - mx), 0.0)
    denom = jnp.sum(ex, axis=-1, keepdims=True)
    probs_ref[...] = ex / denom
    rank = jnp.dot(chosen.astype(jnp.float32), tri_ref[...],
                   preferred_element_type=jnp.float32)
    cols = [jnp.sum(jnp.where(chosen & (rank == k + 1), ex, 0.0),
                    axis=-1, keepdims=True) / denom for k in range(_TOPK)]
    zero = jnp.zeros_like(cols[0])
    pv_ref[...] = jnp.concatenate(cols + [zero] * (8 - _TOPK), axis=-1)


def _router(x2d, gate_w, bias, tri):
    bt = 512
    return pl.pallas_call(
        _router_body,
        grid=(_T // bt,),
        in_specs=[
            pl.BlockSpec((bt, _D), lambda i: (i, 0)),
            pl.BlockSpec((_D, _E), lambda i: (0, 0)),
            pl.BlockSpec((1, _E), lambda i: (0, 0)),
            pl.BlockSpec((_E, _E), lambda i: (0, 0)),
        ],
        out_specs=[
            pl.BlockSpec((bt, _E), lambda i: (i, 0)),
            pl.BlockSpec((bt, 8), lambda i: (i, 0)),
        ],
        out_shape=[
            jax.ShapeDtypeStruct((_T, _E), jnp.float32),
            jax.ShapeDtypeStruct((_T, 8), jnp.float32),
        ],
    )(x2d, gate_w, bias.reshape(1, _E), tri)


def _sc_scatter(src, didx4, epad):
    t, d = src.shape
    per_w = t // _NW
    chunk = 64
    n_chunks = per_w // chunk
    mesh = plsc.VectorSubcoreMesh(core_axis_name="c", subcore_axis_name="s")

    @functools.partial(
        pl.kernel,
        mesh=mesh,
        out_type=jax.ShapeDtypeStruct((epad, d), jnp.float32),
        scratch_types=[
            pltpu.VMEM((chunk, d), jnp.float32),
            pltpu.VMEM((chunk, d), jnp.float32),
            pltpu.VMEM((_TOPK, n_chunks, chunk), jnp.int32),
            pltpu.SemaphoreType.DMA,
            pltpu.SemaphoreType.DMA,
            pltpu.SemaphoreType.DMA,
            pltpu.SemaphoreType.DMA,
        ],
    )
    def sk(src_hbm, didx_hbm, out_hbm, buf0, buf1, idx3, rs0, rs1, ws0, ws1):
        wid = lax.axis_index("s") * 2 + lax.axis_index("c")
        base = pl.multiple_of(wid * per_w, 8)
        bufs = (buf0, buf1)
        rsems = (rs0, rs1)
        wsems = (ws0, ws1)

        def read(c):
            off = pl.multiple_of(base + c * chunk, 8)
            return pltpu.async_copy(
                src_hbm.at[pl.ds(off, chunk)], bufs[c % 2], rsems[c % 2])

        for k in range(_TOPK):
            pltpu.sync_copy(didx_hbm.at[k, wid], idx3.at[k])
        read(0)
        read(1)
        for c in range(n_chunks):
            off = pl.multiple_of(base + c * chunk, 8)
            pltpu.make_async_copy(
                src_hbm.at[pl.ds(off, chunk)], bufs[c % 2],
                rsems[c % 2]).wait()
            handles = [
                pltpu.async_copy(bufs[c % 2], out_hbm.at[idx3.at[k, c]],
                                 wsems[c % 2])
                for k in range(_TOPK)
            ]
            for h in handles:
                h.wait()
            if c + 2 < n_chunks:
                read(c + 2)

    return sk(src, didx4)


def _sc_gather(src, idx, chunk):
    m, d = idx.shape[0], src.shape[1]
    per_w = m // _NW
    n_chunks = per_w // chunk
    mesh = plsc.VectorSubcoreMesh(core_axis_name="c", subcore_axis_name="s")

    @functools.partial(
        pl.kernel,
        mesh=mesh,
        out_type=jax.ShapeDtypeStruct((m, d), jnp.float32),
        scratch_types=[
            pltpu.VMEM((per_w,), jnp.int32),
            pltpu.VMEM((chunk, d), jnp.float32),
            pltpu.VMEM((chunk, d), jnp.float32),
            pltpu.SemaphoreType.DMA,
            pltpu.SemaphoreType.DMA,
        ],
    )
    def gk(src_hbm, idx_hbm, out_hbm, idx_v, buf0, buf1, sem0, sem1):
        wid = lax.axis_index("s") * 2 + lax.axis_index("c")
        base = pl.multiple_of(wid * per_w, 8)
        pltpu.sync_copy(idx_hbm.at[pl.ds(base, per_w)], idx_v)

        def start(j, buf, sem):
            off = pl.multiple_of(j * chunk, 8)
            return pltpu.async_copy(
                src_hbm.at[idx_v.at[pl.ds(off, chunk)]], buf, sem)

        def finish(j, buf, sem):
            ioff = pl.multiple_of(j * chunk, 8)
            pltpu.make_async_copy(
                src_hbm.at[idx_v.at[pl.ds(ioff, chunk)]], buf, sem).wait()
            off = pl.multiple_of(base + j * chunk, 8)
            pltpu.sync_copy(buf, out_hbm.at[pl.ds(off, chunk)])

        start(0, buf0, sem0)

        def body(jj, carry):
            j0 = jj * 2

            @pl.when(j0 + 1 < n_chunks)
            def _():
                start(j0 + 1, buf1, sem1)

            finish(j0, buf0, sem0)

            @pl.when(j0 + 2 < n_chunks)
            def _():
                start(j0 + 2, buf0, sem0)

            @pl.when(j0 + 1 < n_chunks)
            def _():
                finish(j0 + 1, buf1, sem1)

            return carry

        lax.fori_loop(0, (n_chunks + 1) // 2, body, 0)

    return gk(src, idx)


def _expert_body(eot_ref, x_ref, w1_ref, w2_ref, w3_ref, y_ref):
    xb = x_ref[...].astype(jnp.bfloat16)
    h = jnp.dot(xb, w1_ref[0], preferred_element_type=jnp.float32)
    g = h * jax.nn.sigmoid(h)
    v = jnp.dot(xb, w2_ref[0], preferred_element_type=jnp.float32)
    gv = (g * v).astype(jnp.bfloat16)
    y_ref[...] = jnp.dot(gv, w3_ref[0], preferred_element_type=jnp.float32)


def _grouped_experts(exp_tile, xs, w1, w2, w3):
    def wmap(i, eot):
        return (eot[i], 0, 0)

    grid_spec = pltpu.PrefetchScalarGridSpec(
        num_scalar_prefetch=1,
        grid=(_NTILES,),
        in_specs=[
            pl.BlockSpec((_BM, _D), lambda i, eot: (i, 0)),
            pl.BlockSpec((1, _D, _H), wmap),
            pl.BlockSpec((1, _D, _H), wmap),
            pl.BlockSpec((1, _H, _D), wmap),
        ],
        out_specs=pl.BlockSpec((_BM, _D), lambda i, eot: (i, 0)),
    )
    return pl.pallas_call(
        _expert_body,
        grid_spec=grid_spec,
        out_shape=jax.ShapeDtypeStruct((_EPAD, _D), jnp.float32),
    )(exp_tile, xs, w1, w2, w3)


def _shared_body(x_ref, w1_ref, w2_ref, w3_ref, y_ref):
    xb = x_ref[...].astype(jnp.bfloat16)
    h = jnp.dot(xb, w1_ref[...], preferred_element_type=jnp.float32)
    g = h * jax.nn.sigmoid(h)
    v = jnp.dot(xb, w2_ref[...], preferred_element_type=jnp.float32)
    gv = (g * v).astype(jnp.bfloat16)
    y_ref[...] = jnp.dot(gv, w3_ref[...], preferred_element_type=jnp.float32)


def _shared_expert(x2d, sw1, sw2, sw3):
    return pl.pallas_call(
        _shared_body,
        grid=(_T // _BM,),
        in_specs=[
            pl.BlockSpec((_BM, _D), lambda i: (i, 0)),
            pl.BlockSpec((_D, _H), lambda i: (0, 0)),
            pl.BlockSpec((_D, _H), lambda i: (0, 0)),
            pl.BlockSpec((_H, _D), lambda i: (0, 0)),
        ],
        out_specs=pl.BlockSpec((_BM, _D), lambda i: (i, 0)),
        out_shape=jax.ShapeDtypeStruct((_T, _D), jnp.float32),
    )(x2d, sw1, sw2, sw3)


def _combine_body(z_ref, ysh_ref, pv_ref, o_ref):
    z = z_ref[...]
    pv = pv_ref[...]
    acc = ysh_ref[...]
    for k in range(_TOPK):
        acc = acc + z[k] * pv[:, k:k + 1]
    o_ref[...] = acc


def _combine(z, y, pv):
    bc = 512
    return pl.pallas_call(
        _combine_body,
        grid=(_T // bc,),
        in_specs=[
            pl.BlockSpec((_TOPK, bc, _D), lambda i: (0, i, 0)),
            pl.BlockSpec((bc, _D), lambda i: (i, 0)),
            pl.BlockSpec((bc, 8), lambda i: (i, 0)),
        ],
        out_specs=pl.BlockSpec((bc, _D), lambda i: (i, 0)),
        out_shape=jax.ShapeDtypeStruct((_T, _D), jnp.float32),
    )(z, y, pv)


def kernel(x, gate_w, w1, w2, w3, sw1, sw2, sw3, routing_bias):
    b, s, _ = x.shape
    x2d = x.reshape(_T, _D)

    tri = jnp.triu(jnp.ones((_E, _E), jnp.float32))
    probs, pv = _router(x2d, gate_w, routing_bias, tri)

    mask = probs > 0.0
    maski = mask.astype(jnp.int32)
    counts = jnp.sum(maski, axis=0)
    padded = ((counts + _BM - 1) // _BM) * _BM
    ends = jnp.cumsum(padded)
    starts = ends - padded
    rank = jnp.cumsum(maski, axis=0) - 1
    destf = jnp.where(mask, starts[None, :] + rank, 0)

    rank_in_row = jnp.cumsum(maski, axis=1) - 1
    nrow = jnp.sum(maski, axis=1)
    dest4 = [jnp.where(
        nrow > k,
        jnp.sum(jnp.where(mask & (rank_in_row == k), destf, 0), axis=1),
        _LAST) for k in range(_TOPK)]
    didx = jnp.stack(dest4)
    dest_flat = didx.reshape(-1)
    didx4 = didx.reshape(_TOPK, _NW, -1, 64)

    ntiles_e = padded // _BM
    exp_tile = jnp.repeat(jnp.arange(_E, dtype=jnp.int32), ntiles_e,
                          total_repeat_length=_NTILES)

    xs = _sc_scatter(x2d, didx4, _EPAD)
    ysh = _shared_expert(x2d, sw1.astype(jnp.bfloat16),
                         sw2.astype(jnp.bfloat16), sw3.astype(jnp.bfloat16))
    y = _grouped_experts(exp_tile, xs, w1.astype(jnp.bfloat16),
                         w2.astype(jnp.bfloat16), w3.astype(jnp.bfloat16))
    z = _sc_gather(y, dest_flat, chunk=64)
    out2d = _combine(z.reshape(_TOPK, _T, _D), ysh, pv)
    return out2d.reshape(b, s, _D)
